# Initial kernel scaffold; baseline (speedup 1.0000x reference)
import jax
import jax.numpy as jnp
from jax.experimental import pallas as pl


def kernel(instances, bag_label, W1, b1, W2, b2):
    raise NotImplementedError("write your pallas kernel here")


# speedup vs baseline: 2.5407x; 2.5407x over previous
import functools

import jax
import jax.numpy as jnp
from jax.experimental import pallas as pl
from jax.experimental.pallas import tpu as pltpu

N_INST = 20000
D_FEAT = 1024
D_HID = 256
K_SEL = N_INST // 2

BN = 1000
R_PAD = 160
N_PAD = R_PAD * 128


def _mlp_kernel(x_ref, w1_ref, b1_ref, w2_ref, b2_ref, out_ref):
    h = jnp.dot(x_ref[...], w1_ref[...], preferred_element_type=jnp.float32)
    h = jnp.maximum(h + b1_ref[...], 0.0)
    out_ref[...] = (
        jnp.dot(h, w2_ref[...], preferred_element_type=jnp.float32) + b2_ref[...]
    )


def _select_kernel(s_ref, bl_ref, lab_ref):
    s = s_ref[...]
    b = jax.lax.bitcast_convert_type(s, jnp.int32)
    u_i = b ^ ((b >> 31) | jnp.int32(-0x80000000))
    u = jax.lax.bitcast_convert_type(u_i, jnp.uint32)
    row = jax.lax.broadcasted_iota(jnp.int32, (R_PAD, 128), 0)
    col = jax.lax.broadcasted_iota(jnp.int32, (R_PAD, 128), 1)
    flat_idx = row * 128 + col
    u = jnp.where(flat_idx < N_INST, u, jnp.uint32(0))

    kK = jnp.int32(K_SEL)
    npad = jnp.int32(N_PAD - N_INST)

    def body(i, carry):
        t, t2 = carry
        bit = jnp.uint32(1) << (31 - i)
        cand = t | bit
        cnt_ge = jnp.sum((u >= cand).astype(jnp.int32))
        t = jnp.where(cnt_ge >= kK, cand, t)
        cand2 = t2 | bit
        cnt_lt = jnp.sum((u < cand2).astype(jnp.int32)) - npad
        t2 = jnp.where(cnt_lt < kK, cand2, t2)
        return t, t2

    T, T2 = jax.lax.fori_loop(
        0, 32, body, (jnp.uint32(0), jnp.uint32(0)), unroll=True
    )

    G = jnp.sum((u > T).astype(jnp.int32))
    L = jnp.sum((u < T2).astype(jnp.int32)) - npad

    eqT = (u == T).astype(jnp.float32)
    eqT2 = (u == T2).astype(jnp.float32)
    tri_r = (
        jax.lax.broadcasted_iota(jnp.int32, (R_PAD, R_PAD), 0)
        > jax.lax.broadcasted_iota(jnp.int32, (R_PAD, R_PAD), 1)
    ).astype(jnp.float32)
    tri_c = (
        jax.lax.broadcasted_iota(jnp.int32, (128, 128), 0)
        < jax.lax.broadcasted_iota(jnp.int32, (128, 128), 1)
    ).astype(jnp.float32)
    ones = jnp.ones((128, 1), jnp.float32)

    def excl_prefix(m):
        row_tot = jnp.dot(m, ones, preferred_element_type=jnp.float32)
        row_excl = jnp.dot(tri_r, row_tot, preferred_element_type=jnp.float32)
        within = jnp.dot(m, tri_c, preferred_element_type=jnp.float32)
        return row_excl + within

    prefT = excl_prefix(eqT)
    prefT2 = excl_prefix(eqT2)

    in_top = (u > T) | ((u == T) & (prefT < (kK - G).astype(jnp.float32)))
    in_bot = (u < T2) | ((u == T2) & (prefT2 < (kK - L).astype(jnp.float32)))

    top_val = jnp.where(bl_ref[0, 0] != 0.0, jnp.float32(1.0), jnp.float32(0.0))
    lab_ref[...] = jnp.where(in_top & ~in_bot, top_val, jnp.float32(0.0))


@functools.partial(jax.jit, static_argnames=())
def _run(instances, bag_label, W1, b1, W2, b2):
    x = instances[0]
    preds = pl.pallas_call(
        _mlp_kernel,
        grid=(N_INST // BN,),
        in_specs=[
            pl.BlockSpec((BN, D_FEAT), lambda i: (i, 0)),
            pl.BlockSpec((D_FEAT, D_HID), lambda i: (0, 0)),
            pl.BlockSpec((1, D_HID), lambda i: (0, 0)),
            pl.BlockSpec((D_HID, 1), lambda i: (0, 0)),
            pl.BlockSpec((1, 1), lambda i: (0, 0)),
        ],
        out_specs=pl.BlockSpec((BN, 1), lambda i: (i, 0)),
        out_shape=jax.ShapeDtypeStruct((N_INST, 1), jnp.float32),
    )(x, W1, b1.reshape(1, D_HID), W2, b2.reshape(1, 1))

    s_pad = jnp.pad(preds[:, 0], (0, N_PAD - N_INST)).reshape(R_PAD, 128)
    labels = pl.pallas_call(
        _select_kernel,
        in_specs=[
            pl.BlockSpec((R_PAD, 128), lambda: (0, 0)),
            pl.BlockSpec(memory_space=pltpu.SMEM),
        ],
        out_specs=pl.BlockSpec((R_PAD, 128), lambda: (0, 0)),
        out_shape=jax.ShapeDtypeStruct((R_PAD, 128), jnp.float32),
    )(s_pad, bag_label.reshape(1, 1))

    labels = labels.reshape(-1)[:N_INST]
    return preds[None, ...], labels[:, None][None, ...]


def kernel(instances, bag_label, W1, b1, W2, b2):
    return _run(instances, bag_label, W1, b1, W2, b2)


# speedup vs baseline: 2.8193x; 1.1096x over previous
import functools

import jax
import jax.numpy as jnp
from jax.experimental import pallas as pl
from jax.experimental.pallas import tpu as pltpu

N_INST = 20000
D_FEAT = 1024
D_HID = 256
K_SEL = N_INST // 2

BN = 2000
R_PAD = 160
N_PAD = R_PAD * 128


def _mlp_kernel(x_ref, w1_ref, b1_ref, w2_ref, b2_ref, out_ref):
    h = jnp.dot(x_ref[...], w1_ref[...], preferred_element_type=jnp.float32)
    h = jnp.maximum(h + b1_ref[...], 0.0)
    out_ref[...] = (
        jnp.dot(h, w2_ref[...], preferred_element_type=jnp.float32) + b2_ref[...]
    )


def _select_kernel(s_ref, bl_ref, lab_ref):
    s = s_ref[...]
    b = jax.lax.bitcast_convert_type(s, jnp.int32)
    u_i = b ^ ((b >> 31) | jnp.int32(-0x80000000))
    u = jax.lax.bitcast_convert_type(u_i, jnp.uint32)
    row = jax.lax.broadcasted_iota(jnp.int32, (R_PAD, 128), 0)
    col = jax.lax.broadcasted_iota(jnp.int32, (R_PAD, 128), 1)
    flat_idx = row * 128 + col
    u = jnp.where(flat_idx < N_INST, u, jnp.uint32(0))

    kK = jnp.int32(K_SEL)
    npad = jnp.int32(N_PAD - N_INST)

    def body(i, carry):
        t, t2 = carry
        bit = jnp.uint32(1) << (31 - i)
        cand = t | bit
        cnt_ge = jnp.sum((u >= cand).astype(jnp.int32))
        t = jnp.where(cnt_ge >= kK, cand, t)
        cand2 = t2 | bit
        cnt_lt = jnp.sum((u < cand2).astype(jnp.int32)) - npad
        t2 = jnp.where(cnt_lt < kK, cand2, t2)
        return t, t2

    T, T2 = jax.lax.fori_loop(
        0, 32, body, (jnp.uint32(0), jnp.uint32(0)), unroll=True
    )

    G = jnp.sum((u > T).astype(jnp.int32))
    L = jnp.sum((u < T2).astype(jnp.int32)) - npad

    eqT = (u == T).astype(jnp.float32)
    eqT2 = (u == T2).astype(jnp.float32)
    tri_r = (
        jax.lax.broadcasted_iota(jnp.int32, (R_PAD, R_PAD), 0)
        > jax.lax.broadcasted_iota(jnp.int32, (R_PAD, R_PAD), 1)
    ).astype(jnp.float32)
    tri_c = (
        jax.lax.broadcasted_iota(jnp.int32, (128, 128), 0)
        < jax.lax.broadcasted_iota(jnp.int32, (128, 128), 1)
    ).astype(jnp.float32)
    ones = jnp.ones((128, 1), jnp.float32)

    def excl_prefix(m):
        row_tot = jnp.dot(m, ones, preferred_element_type=jnp.float32)
        row_excl = jnp.dot(tri_r, row_tot, preferred_element_type=jnp.float32)
        within = jnp.dot(m, tri_c, preferred_element_type=jnp.float32)
        return row_excl + within

    prefT = excl_prefix(eqT)
    prefT2 = excl_prefix(eqT2)

    in_top = (u > T) | ((u == T) & (prefT < (kK - G).astype(jnp.float32)))
    in_bot = (u < T2) | ((u == T2) & (prefT2 < (kK - L).astype(jnp.float32)))

    top_val = jnp.where(bl_ref[0, 0] != 0.0, jnp.float32(1.0), jnp.float32(0.0))
    lab_ref[...] = jnp.where(in_top & ~in_bot, top_val, jnp.float32(0.0))


@functools.partial(jax.jit, static_argnames=())
def _run(instances, bag_label, W1, b1, W2, b2):
    x = instances[0]
    preds = pl.pallas_call(
        _mlp_kernel,
        grid=(N_INST // BN,),
        in_specs=[
            pl.BlockSpec((BN, D_FEAT), lambda i: (i, 0)),
            pl.BlockSpec((D_FEAT, D_HID), lambda i: (0, 0)),
            pl.BlockSpec((1, D_HID), lambda i: (0, 0)),
            pl.BlockSpec((D_HID, 1), lambda i: (0, 0)),
            pl.BlockSpec((1, 1), lambda i: (0, 0)),
        ],
        out_specs=pl.BlockSpec((BN, 1), lambda i: (i, 0)),
        out_shape=jax.ShapeDtypeStruct((N_INST, 1), jnp.float32),
    )(x, W1, b1.reshape(1, D_HID), W2, b2.reshape(1, 1))

    s_pad = jnp.pad(preds[:, 0], (0, N_PAD - N_INST)).reshape(R_PAD, 128)
    labels = pl.pallas_call(
        _select_kernel,
        in_specs=[
            pl.BlockSpec((R_PAD, 128), lambda: (0, 0)),
            pl.BlockSpec(memory_space=pltpu.SMEM),
        ],
        out_specs=pl.BlockSpec((R_PAD, 128), lambda: (0, 0)),
        out_shape=jax.ShapeDtypeStruct((R_PAD, 128), jnp.float32),
    )(s_pad, bag_label.reshape(1, 1))

    labels = labels.reshape(-1)[:N_INST]
    return preds[None, ...], labels[:, None][None, ...]


def kernel(instances, bag_label, W1, b1, W2, b2):
    return _run(instances, bag_label, W1, b1, W2, b2)
